# baseline (speedup 1.0000x reference)
import functools

import jax
import jax.numpy as jnp
from jax.experimental import pallas as pl
from jax.experimental.pallas import tpu as pltpu


def _glu_kernel(x_ref, wgv_ref, wout_ref, o_ref, h_ref, *, rank):
    @pl.when(pl.program_id(1) == 0)
    def _compute_gate():
        p = jax.lax.dot_general(
            x_ref[...], wgv_ref[...],
            dimension_numbers=(((1,), (1,)), ((), ())),
            preferred_element_type=jnp.float32,
        )
        h_ref[...] = jax.nn.sigmoid(p[:, :rank]) * p[:, rank:]

    o_ref[...] = jax.lax.dot_general(
        h_ref[...], wout_ref[...],
        dimension_numbers=(((1,), (0,)), ((), ())),
        preferred_element_type=jnp.float32,
    )


@functools.partial(jax.jit, static_argnames=())
def kernel(x, W_in, A_log, A_imag, W_dt, W_phase, W_out):
    del A_log, A_imag, W_dt, W_phase
    m, d_model = x.shape
    rank = W_out.shape[1]
    w_gv = W_in[: 2 * rank]
    w_out_t = W_out.T

    tm = 2048
    tn = 512
    grid = (m // tm, d_model // tn)
    return pl.pallas_call(
        functools.partial(_glu_kernel, rank=rank),
        grid=grid,
        in_specs=[
            pl.BlockSpec((tm, d_model), lambda i, j: (i, 0)),
            pl.BlockSpec((2 * rank, d_model), lambda i, j: (0, 0)),
            pl.BlockSpec((rank, tn), lambda i, j: (0, j)),
        ],
        out_specs=pl.BlockSpec((tm, tn), lambda i, j: (i, j)),
        out_shape=jax.ShapeDtypeStruct((m, d_model), jnp.float32),
        scratch_shapes=[pltpu.VMEM((tm, rank), jnp.float32)],
    )(x, w_gv, w_out_t)


# speedup vs baseline: 1.2760x; 1.2760x over previous
import functools

import jax
import jax.numpy as jnp
from jax.experimental import pallas as pl
from jax.experimental.pallas import tpu as pltpu


def _glu_kernel(x_ref, wgv_ref, wout_ref, o_ref, *, rank):
    p = jax.lax.dot_general(
        x_ref[...], wgv_ref[...],
        dimension_numbers=(((1,), (1,)), ((), ())),
        preferred_element_type=jnp.float32,
    )
    h = jax.nn.sigmoid(p[:, :rank]) * p[:, rank:]
    o_ref[...] = jax.lax.dot_general(
        h, wout_ref[...],
        dimension_numbers=(((1,), (0,)), ((), ())),
        preferred_element_type=jnp.float32,
    )


@functools.partial(jax.jit, static_argnames=())
def kernel(x, W_in, A_log, A_imag, W_dt, W_phase, W_out):
    del A_log, A_imag, W_dt, W_phase
    m, d_model = x.shape
    rank = W_out.shape[1]
    w_gv = W_in[: 2 * rank]
    w_out_t = W_out.T

    tm = 2048
    grid = (m // tm,)
    return pl.pallas_call(
        functools.partial(_glu_kernel, rank=rank),
        grid=grid,
        in_specs=[
            pl.BlockSpec((tm, d_model), lambda i: (i, 0)),
            pl.BlockSpec((2 * rank, d_model), lambda i: (0, 0)),
            pl.BlockSpec((rank, d_model), lambda i: (0, 0)),
        ],
        out_specs=pl.BlockSpec((tm, d_model), lambda i: (i, 0)),
        out_shape=jax.ShapeDtypeStruct((m, d_model), jnp.float32),
        compiler_params=pltpu.CompilerParams(
            dimension_semantics=("parallel",),
        ),
    )(x, w_gv, w_out_t)
